# baseline (speedup 1.0000x reference)
import functools

import jax
import jax.numpy as jnp
from jax import lax
from jax.experimental import pallas as pl
from jax.experimental.pallas import tpu as pltpu
from jax.experimental.pallas import tpu_sc as plsc

N = 50000
E = 800000
C = 64

NC = 2
NS = 16
NW = NC * NS
L = 16

R = 1563
P = NW * R
F = 8192
SCH = 6400
UG = 16
SS = F + 288
CAP = 100 * F
G = 128
RZ = 1600
ZR = 64

_MESH = plsc.VectorSubcoreMesh(
    core_axis_name="c", subcore_axis_name="s", num_cores=NC, num_subcores=NS
)

_SC_PARAMS = pltpu.CompilerParams(
    use_tc_tiling_on_sc=False,
    needs_layout_passes=False,
)


def _tile_id():
    return lax.axis_index("s") * NC + lax.axis_index("c")


def _bin_body(src_hbm, dst_hbm, bsrc_hbm, bdst_hbm, cnt_hbm,
              srcv, dstv, sstag, dstag, cbuf):
    t = _tile_id()
    lo = t * R

    def chunk_body(k, carry):
        pltpu.sync_copy(src_hbm.at[pl.ds(k * SCH, SCH)], srcv)
        pltpu.sync_copy(dst_hbm.at[pl.ds(k * SCH, SCH)], dstv)

        def grp16(i2, carry2):
            p_vec, g = carry2
            for ii in range(UG):
                i = i2 * UG + ii
                d = dstv[pl.ds(i * L, L)]
                s = srcv[pl.ds(i * L, L)]
                m = (d >= lo) & (d < lo + R)
                ranks = plsc.cumsum(m.astype(jnp.int32))
                tot = plsc.all_reduce_population_count(m)
                tgt = jnp.where(m, p_vec + ranks - 1, SS - 1)
                plsc.store_scatter(sstag, [tgt], s)
                plsc.store_scatter(dstag, [tgt], d - lo)
                p_vec = p_vec + tot
            p = p_vec[0]
            fl = p >= F

            @pl.when(fl)
            def _():
                pltpu.sync_copy(sstag.at[pl.ds(0, F)],
                                bsrc_hbm.at[t, pl.ds(g * F, F)])
                pltpu.sync_copy(dstag.at[pl.ds(0, F)],
                                bdst_hbm.at[t, pl.ds(g * F, F)])
                for r in range(UG):
                    sstag[pl.ds(r * L, L)] = sstag[pl.ds(F + r * L, L)]
                    dstag[pl.ds(r * L, L)] = dstag[pl.ds(F + r * L, L)]

            p_vec = jnp.where(fl, p_vec - F, p_vec)
            g = g + fl.astype(jnp.int32)
            return (p_vec, g)

        return lax.fori_loop(0, SCH // L // UG, grp16, carry)

    p_vec, g = lax.fori_loop(0, E // SCH, chunk_body,
                             (jnp.zeros((L,), jnp.int32), jnp.int32(0)))
    pltpu.sync_copy(sstag.at[pl.ds(0, F)], bsrc_hbm.at[t, pl.ds(g * F, F)])
    pltpu.sync_copy(dstag.at[pl.ds(0, F)], bdst_hbm.at[t, pl.ds(g * F, F)])
    cnt = g * F + p_vec[0]
    cbuf[...] = jnp.where(lax.iota(jnp.int32, L) == 0, cnt, 0)
    pltpu.sync_copy(cbuf, cnt_hbm.at[t])


_bin_edges = pl.kernel(
    _bin_body,
    out_type=[
        jax.ShapeDtypeStruct((NW, CAP), jnp.int32),
        jax.ShapeDtypeStruct((NW, CAP), jnp.int32),
        jax.ShapeDtypeStruct((NW, L), jnp.int32),
    ],
    mesh=_MESH,
    compiler_params=_SC_PARAMS,
    scratch_types=[
        pltpu.VMEM((SCH,), jnp.int32),
        pltpu.VMEM((SCH,), jnp.int32),
        pltpu.VMEM((SS,), jnp.int32),
        pltpu.VMEM((SS,), jnp.int32),
        pltpu.VMEM((L,), jnp.int32),
    ],
)


def _agg_body(tab_hbm, bsrc_hbm, bdst_hbm, cnt_hbm, out_hbm,
              idxv, ldv, rows, sacc, zbuf, cntv, gsems, ssems):
    t = _tile_id()
    sid = lax.axis_index("s")
    base_row = sid * RZ
    pltpu.sync_copy(cnt_hbm.at[t], cntv)
    cnt = cntv[pl.ds(0, L)][0]

    def zrow(i, _):
        for j in range(C // L):
            zbuf[i, pl.ds(j * L, L)] = jnp.zeros((L,), jnp.float32)
        return 0

    lax.fori_loop(0, ZR, zrow, 0)

    def zcopy(i, _):
        pltpu.sync_copy(zbuf, sacc.at[pl.ds(base_row + i * ZR, ZR)])
        return 0

    lax.fori_loop(0, RZ // ZR, zcopy, 0)

    nch = (cnt + G - 1) // G
    nfull = cnt // G

    def stage(k):
        b = k % 2
        pltpu.sync_copy(bsrc_hbm.at[t, pl.ds(k * G, G)], idxv.at[b])
        pltpu.sync_copy(bdst_hbm.at[t, pl.ds(k * G, G)], ldv.at[b])

        @pl.when(k == nfull)
        def _():
            def fix(i, _):
                ev = k * G + i * L + lax.iota(jnp.int32, L)
                mm = ev < cnt
                s = idxv[b, pl.ds(i * L, L)]
                idxv[b, pl.ds(i * L, L)] = jnp.where(
                    mm, jnp.clip(s, 0, N - 1), 0)
                d = ldv[b, pl.ds(i * L, L)]
                ldv[b, pl.ds(i * L, L)] = jnp.where(
                    mm, jnp.clip(d, 0, R - 1), R)
                return 0

            lax.fori_loop(0, G // L, fix, 0)

        def shift(i, _):
            ldv[b, pl.ds(i * L, L)] = ldv[b, pl.ds(i * L, L)] + base_row
            return 0

        lax.fori_loop(0, G // L, shift, 0)
        pltpu.make_async_copy(tab_hbm.at[idxv.at[b]], rows.at[b],
                              gsems.at[b]).start()

    def scat(b, action):
        d = pltpu.make_async_copy(rows.at[b], sacc.at[ldv.at[b]],
                                  ssems.at[b])
        if action == "start":
            d.start(add=True)
        else:
            d.wait()

    @pl.when(nch > 0)
    def _():
        stage(0)

    def chunk(k, _):
        b = k % 2
        pltpu.make_async_copy(tab_hbm.at[idxv.at[b]], rows.at[b],
                              gsems.at[b]).wait()
        scat(b, "start")

        @pl.when(k + 1 < nch)
        def _():
            @pl.when(k >= 1)
            def _():
                scat(1 - b, "wait")

            stage(k + 1)

        return 0

    lax.fori_loop(0, nch, chunk, 0)

    @pl.when(nch >= 2)
    def _():
        scat(nch % 2, "wait")

    @pl.when(nch >= 1)
    def _():
        scat((nch - 1) % 2, "wait")

    pltpu.sync_copy(sacc.at[pl.ds(base_row, R)],
                    out_hbm.at[pl.ds(t * R, R)])


def _make_agg(n_tab):
    return pl.kernel(
        _agg_body,
        out_type=jax.ShapeDtypeStruct((P, C), jnp.float32),
        mesh=_MESH,
        compiler_params=_SC_PARAMS,
        scratch_types=[
            pltpu.VMEM((2, G), jnp.int32),
            pltpu.VMEM((2, G), jnp.int32),
            pltpu.VMEM((2, G, C), jnp.float32),
            pltpu.VMEM_SHARED((NS * RZ, C), jnp.float32),
            pltpu.VMEM((ZR, C), jnp.float32),
            pltpu.VMEM((L,), jnp.int32),
            pltpu.SemaphoreType.DMA((2,)),
            pltpu.SemaphoreType.DMA((2,)),
        ],
    )


_agg_n = _make_agg(N)
_agg_p = _make_agg(P)


BT = 4168
GRID = P // BT

_row_spec = pl.BlockSpec((BT, C), lambda i: (i, 0))
_w_spec = pl.BlockSpec((C, C), lambda i: (0, 0))
_b_spec = pl.BlockSpec((1, C), lambda i: (0, 0))
_f_spec = pl.BlockSpec((BT, 1), lambda i: (i, 0))
_s_spec = pl.BlockSpec(memory_space=pltpu.SMEM)


def _mm_relu_body(a_ref, w_ref, b_ref, o_ref):
    o_ref[...] = jnp.maximum(
        jnp.dot(a_ref[...], w_ref[...],
                preferred_element_type=jnp.float32) + b_ref[...], 0.0)


_mm_relu = pl.pallas_call(
    _mm_relu_body,
    grid=(GRID,),
    in_specs=[_row_spec, _w_spec, _b_spec],
    out_specs=_row_spec,
    out_shape=jax.ShapeDtypeStruct((P, C), jnp.float32),
)


def _res_body(p_ref, a_ref, w_ref, b_ref, o_ref):
    o_ref[...] = jnp.maximum(
        p_ref[...] + jnp.dot(a_ref[...], w_ref[...],
                             preferred_element_type=jnp.float32)
        + b_ref[...], 0.0)


_res = pl.pallas_call(
    _res_body,
    grid=(GRID,),
    in_specs=[_row_spec, _row_spec, _w_spec, _b_spec],
    out_specs=_row_spec,
    out_shape=jax.ShapeDtypeStruct((P, C), jnp.float32),
)


def _cls1_body(a_ref, wc_ref, bc_ref, f_ref, mx_ref, top_ref, m_s, t_s):
    i = pl.program_id(0)
    f = jnp.sum(a_ref[...] * wc_ref[...], axis=1, keepdims=True) + bc_ref[0, 0]
    f_ref[...] = f
    rid = i * BT + lax.broadcasted_iota(jnp.int32, (BT, 1), 0)
    fm = jnp.where(rid < N, f, -jnp.inf)
    bmx = jnp.max(fm)
    btop = jnp.min(jnp.where(fm == bmx, rid, P))

    @pl.when(i == 0)
    def _():
        m_s[0] = -jnp.inf
        t_s[0] = P

    @pl.when(bmx > m_s[0])
    def _():
        m_s[0] = bmx
        t_s[0] = btop

    @pl.when(i == GRID - 1)
    def _():
        mx_ref[0, 0] = m_s[0]
        top_ref[0, 0] = t_s[0]


_cls1 = pl.pallas_call(
    _cls1_body,
    grid=(GRID,),
    in_specs=[_row_spec, _b_spec, _s_spec],
    out_specs=[_f_spec, _s_spec, _s_spec],
    out_shape=[
        jax.ShapeDtypeStruct((P, 1), jnp.float32),
        jax.ShapeDtypeStruct((1, 1), jnp.float32),
        jax.ShapeDtypeStruct((1, 1), jnp.int32),
    ],
    scratch_shapes=[
        pltpu.SMEM((1,), jnp.float32),
        pltpu.SMEM((1,), jnp.int32),
    ],
)


def _cls2_body(f_ref, o_ref, mx_ref, top_ref, keep_ref, opr_ref):
    i = pl.program_id(0)
    rid = i * BT + lax.broadcasted_iota(jnp.int32, (BT, 1), 0)
    keep = (f_ref[...] > 0) | ((mx_ref[0, 0] < 0) & (rid == top_ref[0, 0]))
    keep = keep & (rid < N)
    keep_ref[...] = keep.astype(jnp.int32)
    opr_ref[...] = o_ref[...] * keep.astype(jnp.float32)


_cls2 = pl.pallas_call(
    _cls2_body,
    grid=(GRID,),
    in_specs=[_f_spec, _row_spec, _s_spec, _s_spec],
    out_specs=[_f_spec, _row_spec],
    out_shape=[
        jax.ShapeDtypeStruct((P, 1), jnp.int32),
        jax.ShapeDtypeStruct((P, C), jnp.float32),
    ],
)


def kernel(x, edge_index, target_label, W1, b1, Wa0, ba0, Wb0, bb0,
           Wa1, ba1, Wb1, bb1, Wc, bc):
    src = edge_index[0].astype(jnp.int32)
    dst = edge_index[1].astype(jnp.int32)
    bsrc, bdst, cnts = _bin_edges(src, dst)

    def agg_x(tab):
        return _agg_n(tab, bsrc, bdst, cnts)

    def agg_p(tab):
        return _agg_p(tab, bsrc, bdst, cnts)

    out = _mm_relu(agg_x(x), W1, b1.reshape(1, C))
    for (Wa, ba, Wb, bb) in ((Wa0, ba0, Wb0, bb0), (Wa1, ba1, Wb1, bb1)):
        h = _mm_relu(agg_p(out), Wa, ba.reshape(1, C))
        out = _res(out, agg_p(h), Wb, bb.reshape(1, C))
    ocls, mx, top = _cls1(agg_p(out), Wc.reshape(1, C), bc.reshape(1, 1))
    keep, opr = _cls2(ocls, out, mx, top)
    return (opr[:N], ocls[:N], target_label, keep[:N, 0].astype(bool))


# speedup vs baseline: 1.2465x; 1.2465x over previous
import functools

import jax
import jax.numpy as jnp
from jax import lax
from jax.experimental import pallas as pl
from jax.experimental.pallas import tpu as pltpu
from jax.experimental.pallas import tpu_sc as plsc

N = 50000
E = 800000
C = 64

NC = 2
NS = 16
NW = NC * NS
L = 16

R = 1563
P = NW * R
F = 8192
SCH = 6400
UG = 16
SS = F + 288
CAP = 100 * F
G = 160
RZ = 1600
ZR = 64

_MESH = plsc.VectorSubcoreMesh(
    core_axis_name="c", subcore_axis_name="s", num_cores=NC, num_subcores=NS
)

_SC_PARAMS = pltpu.CompilerParams(
    use_tc_tiling_on_sc=False,
    needs_layout_passes=False,
)


def _tile_id():
    return lax.axis_index("s") * NC + lax.axis_index("c")


def _bin_body(src_hbm, dst_hbm, bsrc_hbm, bdst_hbm, cnt_hbm,
              srcv, dstv, sstag, dstag, cbuf):
    t = _tile_id()
    lo = t * R

    def chunk_body(k, carry):
        pltpu.sync_copy(src_hbm.at[pl.ds(k * SCH, SCH)], srcv)
        pltpu.sync_copy(dst_hbm.at[pl.ds(k * SCH, SCH)], dstv)

        def grp16(i2, carry2):
            p_vec, g = carry2
            for ii in range(UG):
                i = i2 * UG + ii
                d = dstv[pl.ds(i * L, L)]
                s = srcv[pl.ds(i * L, L)]
                m = (d >= lo) & (d < lo + R)
                ranks = plsc.cumsum(m.astype(jnp.int32))
                tot = plsc.all_reduce_population_count(m)
                tgt = jnp.where(m, p_vec + ranks - 1, SS - 1)
                plsc.store_scatter(sstag, [tgt], s)
                plsc.store_scatter(dstag, [tgt], d - lo)
                p_vec = p_vec + tot
            p = p_vec[0]
            fl = p >= F

            @pl.when(fl)
            def _():
                pltpu.sync_copy(sstag.at[pl.ds(0, F)],
                                bsrc_hbm.at[t, pl.ds(g * F, F)])
                pltpu.sync_copy(dstag.at[pl.ds(0, F)],
                                bdst_hbm.at[t, pl.ds(g * F, F)])
                for r in range(UG):
                    sstag[pl.ds(r * L, L)] = sstag[pl.ds(F + r * L, L)]
                    dstag[pl.ds(r * L, L)] = dstag[pl.ds(F + r * L, L)]

            p_vec = jnp.where(fl, p_vec - F, p_vec)
            g = g + fl.astype(jnp.int32)
            return (p_vec, g)

        return lax.fori_loop(0, SCH // L // UG, grp16, carry)

    p_vec, g = lax.fori_loop(0, E // SCH, chunk_body,
                             (jnp.zeros((L,), jnp.int32), jnp.int32(0)))
    pltpu.sync_copy(sstag.at[pl.ds(0, F)], bsrc_hbm.at[t, pl.ds(g * F, F)])
    pltpu.sync_copy(dstag.at[pl.ds(0, F)], bdst_hbm.at[t, pl.ds(g * F, F)])
    cnt = g * F + p_vec[0]
    cbuf[...] = jnp.where(lax.iota(jnp.int32, L) == 0, cnt, 0)
    pltpu.sync_copy(cbuf, cnt_hbm.at[t])


_bin_edges = pl.kernel(
    _bin_body,
    out_type=[
        jax.ShapeDtypeStruct((NW, CAP), jnp.int32),
        jax.ShapeDtypeStruct((NW, CAP), jnp.int32),
        jax.ShapeDtypeStruct((NW, L), jnp.int32),
    ],
    mesh=_MESH,
    compiler_params=_SC_PARAMS,
    scratch_types=[
        pltpu.VMEM((SCH,), jnp.int32),
        pltpu.VMEM((SCH,), jnp.int32),
        pltpu.VMEM((SS,), jnp.int32),
        pltpu.VMEM((SS,), jnp.int32),
        pltpu.VMEM((L,), jnp.int32),
    ],
)


def _agg_body(tab_hbm, bsrc_hbm, bdst_hbm, cnt_hbm, out_hbm,
              idxv, ldv, rows, sacc, zbuf, cntv, gsems):
    t = _tile_id()
    sid = lax.axis_index("s")
    base_row = sid * RZ
    pltpu.sync_copy(cnt_hbm.at[t], cntv)
    cnt = cntv[pl.ds(0, L)][0]

    def zrow(i, _):
        for j in range(C // L):
            zbuf[i, pl.ds(j * L, L)] = jnp.zeros((L,), jnp.float32)
        return 0

    lax.fori_loop(0, ZR, zrow, 0)

    def zcopy(i, _):
        pltpu.sync_copy(zbuf, sacc.at[pl.ds(base_row + i * ZR, ZR)])
        return 0

    lax.fori_loop(0, RZ // ZR, zcopy, 0)

    nch = (cnt + G - 1) // G
    nfull = cnt // G

    def stage(k):
        b = k % 2
        pltpu.sync_copy(bsrc_hbm.at[t, pl.ds(k * G, G)], idxv.at[b])
        pltpu.sync_copy(bdst_hbm.at[t, pl.ds(k * G, G)], ldv.at[b])

        @pl.when(k == nfull)
        def _():
            def fix(i, _):
                ev = k * G + i * L + lax.iota(jnp.int32, L)
                mm = ev < cnt
                s = idxv[b, pl.ds(i * L, L)]
                idxv[b, pl.ds(i * L, L)] = jnp.where(
                    mm, jnp.clip(s, 0, N - 1), 0)
                d = ldv[b, pl.ds(i * L, L)]
                ldv[b, pl.ds(i * L, L)] = jnp.where(
                    mm, jnp.clip(d, 0, R - 1), R)
                return 0

            lax.fori_loop(0, G // L, fix, 0)

        def shift(i, _):
            ldv[b, pl.ds(i * L, L)] = ldv[b, pl.ds(i * L, L)] + base_row
            return 0

        lax.fori_loop(0, G // L, shift, 0)
        pltpu.make_async_copy(tab_hbm.at[idxv.at[b]], rows.at[b],
                              gsems.at[b]).start()

    @pl.when(nch > 0)
    def _():
        stage(0)

    def chunk(k, _):
        b = k % 2

        @pl.when(k + 1 < nch)
        def _():
            stage(k + 1)

        pltpu.make_async_copy(tab_hbm.at[idxv.at[b]], rows.at[b],
                              gsems.at[b]).wait()
        pltpu.sync_copy(rows.at[b], sacc.at[ldv.at[b]], add=True)
        return 0

    lax.fori_loop(0, nch, chunk, 0)
    pltpu.sync_copy(sacc.at[pl.ds(base_row, R)],
                    out_hbm.at[pl.ds(t * R, R)])


def _make_agg(n_tab):
    return pl.kernel(
        _agg_body,
        out_type=jax.ShapeDtypeStruct((P, C), jnp.float32),
        mesh=_MESH,
        compiler_params=_SC_PARAMS,
        scratch_types=[
            pltpu.VMEM((2, G), jnp.int32),
            pltpu.VMEM((2, G), jnp.int32),
            pltpu.VMEM((2, G, C), jnp.float32),
            pltpu.VMEM_SHARED((NS * RZ, C), jnp.float32),
            pltpu.VMEM((ZR, C), jnp.float32),
            pltpu.VMEM((L,), jnp.int32),
            pltpu.SemaphoreType.DMA((2,)),
        ],
    )


_agg_n = _make_agg(N)
_agg_p = _make_agg(P)


BT = 4168
GRID = P // BT

_row_spec = pl.BlockSpec((BT, C), lambda i: (i, 0))
_w_spec = pl.BlockSpec((C, C), lambda i: (0, 0))
_b_spec = pl.BlockSpec((1, C), lambda i: (0, 0))
_f_spec = pl.BlockSpec((BT, 1), lambda i: (i, 0))
_s_spec = pl.BlockSpec(memory_space=pltpu.SMEM)


def _mm_relu_body(a_ref, w_ref, b_ref, o_ref):
    o_ref[...] = jnp.maximum(
        jnp.dot(a_ref[...], w_ref[...],
                preferred_element_type=jnp.float32) + b_ref[...], 0.0)


_mm_relu = pl.pallas_call(
    _mm_relu_body,
    grid=(GRID,),
    in_specs=[_row_spec, _w_spec, _b_spec],
    out_specs=_row_spec,
    out_shape=jax.ShapeDtypeStruct((P, C), jnp.float32),
)


def _res_body(p_ref, a_ref, w_ref, b_ref, o_ref):
    o_ref[...] = jnp.maximum(
        p_ref[...] + jnp.dot(a_ref[...], w_ref[...],
                             preferred_element_type=jnp.float32)
        + b_ref[...], 0.0)


_res = pl.pallas_call(
    _res_body,
    grid=(GRID,),
    in_specs=[_row_spec, _row_spec, _w_spec, _b_spec],
    out_specs=_row_spec,
    out_shape=jax.ShapeDtypeStruct((P, C), jnp.float32),
)


def _cls1_body(a_ref, wc_ref, bc_ref, f_ref, mx_ref, top_ref, m_s, t_s):
    i = pl.program_id(0)
    f = jnp.sum(a_ref[...] * wc_ref[...], axis=1, keepdims=True) + bc_ref[0, 0]
    f_ref[...] = f
    rid = i * BT + lax.broadcasted_iota(jnp.int32, (BT, 1), 0)
    fm = jnp.where(rid < N, f, -jnp.inf)
    bmx = jnp.max(fm)
    btop = jnp.min(jnp.where(fm == bmx, rid, P))

    @pl.when(i == 0)
    def _():
        m_s[0] = -jnp.inf
        t_s[0] = P

    @pl.when(bmx > m_s[0])
    def _():
        m_s[0] = bmx
        t_s[0] = btop

    @pl.when(i == GRID - 1)
    def _():
        mx_ref[0, 0] = m_s[0]
        top_ref[0, 0] = t_s[0]


_cls1 = pl.pallas_call(
    _cls1_body,
    grid=(GRID,),
    in_specs=[_row_spec, _b_spec, _s_spec],
    out_specs=[_f_spec, _s_spec, _s_spec],
    out_shape=[
        jax.ShapeDtypeStruct((P, 1), jnp.float32),
        jax.ShapeDtypeStruct((1, 1), jnp.float32),
        jax.ShapeDtypeStruct((1, 1), jnp.int32),
    ],
    scratch_shapes=[
        pltpu.SMEM((1,), jnp.float32),
        pltpu.SMEM((1,), jnp.int32),
    ],
)


def _cls2_body(f_ref, o_ref, mx_ref, top_ref, keep_ref, opr_ref):
    i = pl.program_id(0)
    rid = i * BT + lax.broadcasted_iota(jnp.int32, (BT, 1), 0)
    keep = (f_ref[...] > 0) | ((mx_ref[0, 0] < 0) & (rid == top_ref[0, 0]))
    keep = keep & (rid < N)
    keep_ref[...] = keep.astype(jnp.int32)
    opr_ref[...] = o_ref[...] * keep.astype(jnp.float32)


_cls2 = pl.pallas_call(
    _cls2_body,
    grid=(GRID,),
    in_specs=[_f_spec, _row_spec, _s_spec, _s_spec],
    out_specs=[_f_spec, _row_spec],
    out_shape=[
        jax.ShapeDtypeStruct((P, 1), jnp.int32),
        jax.ShapeDtypeStruct((P, C), jnp.float32),
    ],
)


def kernel(x, edge_index, target_label, W1, b1, Wa0, ba0, Wb0, bb0,
           Wa1, ba1, Wb1, bb1, Wc, bc):
    src = edge_index[0].astype(jnp.int32)
    dst = edge_index[1].astype(jnp.int32)
    bsrc, bdst, cnts = _bin_edges(src, dst)

    def agg_x(tab):
        return _agg_n(tab, bsrc, bdst, cnts)

    def agg_p(tab):
        return _agg_p(tab, bsrc, bdst, cnts)

    out = _mm_relu(agg_x(x), W1, b1.reshape(1, C))
    for (Wa, ba, Wb, bb) in ((Wa0, ba0, Wb0, bb0), (Wa1, ba1, Wb1, bb1)):
        h = _mm_relu(agg_p(out), Wa, ba.reshape(1, C))
        out = _res(out, agg_p(h), Wb, bb.reshape(1, C))
    ocls, mx, top = _cls1(agg_p(out), Wc.reshape(1, C), bc.reshape(1, 1))
    keep, opr = _cls2(ocls, out, mx, top)
    return (opr[:N], ocls[:N], target_label, keep[:N, 0].astype(bool))


# speedup vs baseline: 1.3658x; 1.0957x over previous
import functools

import jax
import jax.numpy as jnp
from jax import lax
from jax.experimental import pallas as pl
from jax.experimental.pallas import tpu as pltpu
from jax.experimental.pallas import tpu_sc as plsc

N = 50000
E = 800000
C = 64

NC = 2
NS = 16
NW = NC * NS
L = 16

R = 1563
P = NW * R
F = 8192
SCH = 6400
UG = 16
SS = F + 288
CAP = 100 * F
G = 160
IB = 4
BG = IB * G
RZ = 1600
ZR = 64

_MESH = plsc.VectorSubcoreMesh(
    core_axis_name="c", subcore_axis_name="s", num_cores=NC, num_subcores=NS
)

_SC_PARAMS = pltpu.CompilerParams(
    use_tc_tiling_on_sc=False,
    needs_layout_passes=False,
)


def _tile_id():
    return lax.axis_index("s") * NC + lax.axis_index("c")


def _bin_body(src_hbm, dst_hbm, bsrc_hbm, bdst_hbm, cnt_hbm,
              srcv, dstv, sstag, dstag, cbuf):
    t = _tile_id()
    lo = t * R

    def chunk_body(k, carry):
        pltpu.sync_copy(src_hbm.at[pl.ds(k * SCH, SCH)], srcv)
        pltpu.sync_copy(dst_hbm.at[pl.ds(k * SCH, SCH)], dstv)

        def grp16(i2, carry2):
            p_vec, g = carry2
            for ii in range(UG):
                i = i2 * UG + ii
                d = dstv[pl.ds(i * L, L)]
                s = srcv[pl.ds(i * L, L)]
                m = (d >= lo) & (d < lo + R)
                ranks = plsc.cumsum(m.astype(jnp.int32))
                tot = plsc.all_reduce_population_count(m)
                tgt = jnp.where(m, p_vec + ranks - 1, SS - 1)
                plsc.store_scatter(sstag, [tgt], s)
                plsc.store_scatter(dstag, [tgt], d - lo)
                p_vec = p_vec + tot
            p = p_vec[0]
            fl = p >= F

            @pl.when(fl)
            def _():
                pltpu.sync_copy(sstag.at[pl.ds(0, F)],
                                bsrc_hbm.at[t, pl.ds(g * F, F)])
                pltpu.sync_copy(dstag.at[pl.ds(0, F)],
                                bdst_hbm.at[t, pl.ds(g * F, F)])
                for r in range(UG):
                    sstag[pl.ds(r * L, L)] = sstag[pl.ds(F + r * L, L)]
                    dstag[pl.ds(r * L, L)] = dstag[pl.ds(F + r * L, L)]

            p_vec = jnp.where(fl, p_vec - F, p_vec)
            g = g + fl.astype(jnp.int32)
            return (p_vec, g)

        return lax.fori_loop(0, SCH // L // UG, grp16, carry)

    p_vec, g = lax.fori_loop(0, E // SCH, chunk_body,
                             (jnp.zeros((L,), jnp.int32), jnp.int32(0)))
    pltpu.sync_copy(sstag.at[pl.ds(0, F)], bsrc_hbm.at[t, pl.ds(g * F, F)])
    pltpu.sync_copy(dstag.at[pl.ds(0, F)], bdst_hbm.at[t, pl.ds(g * F, F)])
    cnt = g * F + p_vec[0]
    cbuf[...] = jnp.where(lax.iota(jnp.int32, L) == 0, cnt, 0)
    pltpu.sync_copy(cbuf, cnt_hbm.at[t])


_bin_edges = pl.kernel(
    _bin_body,
    out_type=[
        jax.ShapeDtypeStruct((NW, CAP), jnp.int32),
        jax.ShapeDtypeStruct((NW, CAP), jnp.int32),
        jax.ShapeDtypeStruct((NW, L), jnp.int32),
    ],
    mesh=_MESH,
    compiler_params=_SC_PARAMS,
    scratch_types=[
        pltpu.VMEM((SCH,), jnp.int32),
        pltpu.VMEM((SCH,), jnp.int32),
        pltpu.VMEM((SS,), jnp.int32),
        pltpu.VMEM((SS,), jnp.int32),
        pltpu.VMEM((L,), jnp.int32),
    ],
)


def _agg_body(tab_hbm, bsrc_hbm, bdst_hbm, cnt_hbm, out_hbm,
              idxB, ldvB, rows, sacc, zbuf, cntv, gsems, isems, dsems):
    t = _tile_id()
    sid = lax.axis_index("s")
    base_row = sid * RZ
    pltpu.sync_copy(cnt_hbm.at[t], cntv)
    cnt = cntv[pl.ds(0, L)][0]

    def zrow(i, _):
        for j in range(C // L):
            zbuf[i, pl.ds(j * L, L)] = jnp.zeros((L,), jnp.float32)
        return 0

    lax.fori_loop(0, ZR, zrow, 0)

    def zcopy(i, _):
        pltpu.sync_copy(zbuf, sacc.at[pl.ds(base_row + i * ZR, ZR)])
        return 0

    lax.fori_loop(0, RZ // ZR, zcopy, 0)

    nch = (cnt + G - 1) // G
    nblk = (nch + IB - 1) // IB
    kb_fix = cnt // BG

    def load_block(kb):
        bb = kb % 3
        pltpu.make_async_copy(bsrc_hbm.at[t, pl.ds(kb * IB, IB)],
                              idxB.at[bb], isems.at[bb]).start()
        pltpu.make_async_copy(bdst_hbm.at[t, pl.ds(kb * IB, IB)],
                              ldvB.at[bb], dsems.at[bb]).start()

    def ready_block(kb):
        bb = kb % 3
        pltpu.make_async_copy(bsrc_hbm.at[t, pl.ds(kb * IB, IB)],
                              idxB.at[bb], isems.at[bb]).wait()
        pltpu.make_async_copy(bdst_hbm.at[t, pl.ds(kb * IB, IB)],
                              ldvB.at[bb], dsems.at[bb]).wait()

        @pl.when(kb == kb_fix)
        def _():
            for jj in range(IB):
                def fix(i, _):
                    ev = kb * BG + jj * G + i * L + lax.iota(jnp.int32, L)
                    mm = ev < cnt
                    s = idxB[bb, jj, pl.ds(i * L, L)]
                    idxB[bb, jj, pl.ds(i * L, L)] = jnp.where(
                        mm, jnp.clip(s, 0, N - 1), 0)
                    d = ldvB[bb, jj, pl.ds(i * L, L)]
                    ldvB[bb, jj, pl.ds(i * L, L)] = jnp.where(
                        mm, jnp.clip(d, 0, R - 1), R)
                    return 0

                lax.fori_loop(0, G // L, fix, 0)

        for jj in range(IB):
            def shift(i, _):
                ldvB[bb, jj, pl.ds(i * L, L)] = (
                    ldvB[bb, jj, pl.ds(i * L, L)] + base_row)
                return 0

            lax.fori_loop(0, G // L, shift, 0)

    def start_gather(k):
        b = k % 2
        kb = k // IB
        pltpu.make_async_copy(tab_hbm.at[idxB.at[kb % 3, k - kb * IB]],
                              rows.at[b], gsems.at[b]).start()

    @pl.when(nch > 0)
    def _():
        load_block(0)
        ready_block(0)

        @pl.when(nblk > 1)
        def _():
            load_block(1)

        start_gather(0)

    def chunk(k, _):
        b = k % 2
        kb = k // IB
        j = k - kb * IB
        pltpu.make_async_copy(tab_hbm.at[idxB.at[kb % 3, j]],
                              rows.at[b], gsems.at[b]).wait()

        @pl.when(k + 1 < nch)
        def _():
            @pl.when(j == IB - 1)
            def _():
                ready_block(kb + 1)

                @pl.when(kb + 2 < nblk)
                def _():
                    load_block(kb + 2)

            start_gather(k + 1)

        pltpu.sync_copy(rows.at[b], sacc.at[ldvB.at[kb % 3, j]], add=True)
        return 0

    lax.fori_loop(0, nch, chunk, 0)
    pltpu.sync_copy(sacc.at[pl.ds(base_row, R)],
                    out_hbm.at[pl.ds(t * R, R)])


def _make_agg(n_tab):
    return pl.kernel(
        _agg_body,
        out_type=jax.ShapeDtypeStruct((P, C), jnp.float32),
        mesh=_MESH,
        compiler_params=_SC_PARAMS,
        scratch_types=[
            pltpu.VMEM((3, IB, G), jnp.int32),
            pltpu.VMEM((3, IB, G), jnp.int32),
            pltpu.VMEM((2, G, C), jnp.float32),
            pltpu.VMEM_SHARED((NS * RZ, C), jnp.float32),
            pltpu.VMEM((ZR, C), jnp.float32),
            pltpu.VMEM((L,), jnp.int32),
            pltpu.SemaphoreType.DMA((2,)),
            pltpu.SemaphoreType.DMA((3,)),
            pltpu.SemaphoreType.DMA((3,)),
        ],
    )


_agg_n = _make_agg(N)
_agg_p = _make_agg(P)


BT = 4168
GRID = P // BT

_row_spec = pl.BlockSpec((BT, C), lambda i: (i, 0))
_w_spec = pl.BlockSpec((C, C), lambda i: (0, 0))
_b_spec = pl.BlockSpec((1, C), lambda i: (0, 0))
_f_spec = pl.BlockSpec((BT, 1), lambda i: (i, 0))
_s_spec = pl.BlockSpec(memory_space=pltpu.SMEM)


def _mm_relu_body(a_ref, w_ref, b_ref, o_ref):
    o_ref[...] = jnp.maximum(
        jnp.dot(a_ref[...], w_ref[...],
                preferred_element_type=jnp.float32) + b_ref[...], 0.0)


_mm_relu = pl.pallas_call(
    _mm_relu_body,
    grid=(GRID,),
    in_specs=[_row_spec, _w_spec, _b_spec],
    out_specs=_row_spec,
    out_shape=jax.ShapeDtypeStruct((P, C), jnp.float32),
)


def _res_body(p_ref, a_ref, w_ref, b_ref, o_ref):
    o_ref[...] = jnp.maximum(
        p_ref[...] + jnp.dot(a_ref[...], w_ref[...],
                             preferred_element_type=jnp.float32)
        + b_ref[...], 0.0)


_res = pl.pallas_call(
    _res_body,
    grid=(GRID,),
    in_specs=[_row_spec, _row_spec, _w_spec, _b_spec],
    out_specs=_row_spec,
    out_shape=jax.ShapeDtypeStruct((P, C), jnp.float32),
)


def _cls1_body(a_ref, wc_ref, bc_ref, f_ref, mx_ref, top_ref, m_s, t_s):
    i = pl.program_id(0)
    f = jnp.sum(a_ref[...] * wc_ref[...], axis=1, keepdims=True) + bc_ref[0, 0]
    f_ref[...] = f
    rid = i * BT + lax.broadcasted_iota(jnp.int32, (BT, 1), 0)
    fm = jnp.where(rid < N, f, -jnp.inf)
    bmx = jnp.max(fm)
    btop = jnp.min(jnp.where(fm == bmx, rid, P))

    @pl.when(i == 0)
    def _():
        m_s[0] = -jnp.inf
        t_s[0] = P

    @pl.when(bmx > m_s[0])
    def _():
        m_s[0] = bmx
        t_s[0] = btop

    @pl.when(i == GRID - 1)
    def _():
        mx_ref[0, 0] = m_s[0]
        top_ref[0, 0] = t_s[0]


_cls1 = pl.pallas_call(
    _cls1_body,
    grid=(GRID,),
    in_specs=[_row_spec, _b_spec, _s_spec],
    out_specs=[_f_spec, _s_spec, _s_spec],
    out_shape=[
        jax.ShapeDtypeStruct((P, 1), jnp.float32),
        jax.ShapeDtypeStruct((1, 1), jnp.float32),
        jax.ShapeDtypeStruct((1, 1), jnp.int32),
    ],
    scratch_shapes=[
        pltpu.SMEM((1,), jnp.float32),
        pltpu.SMEM((1,), jnp.int32),
    ],
)


def _cls2_body(f_ref, o_ref, mx_ref, top_ref, keep_ref, opr_ref):
    i = pl.program_id(0)
    rid = i * BT + lax.broadcasted_iota(jnp.int32, (BT, 1), 0)
    keep = (f_ref[...] > 0) | ((mx_ref[0, 0] < 0) & (rid == top_ref[0, 0]))
    keep = keep & (rid < N)
    keep_ref[...] = keep.astype(jnp.int32)
    opr_ref[...] = o_ref[...] * keep.astype(jnp.float32)


_cls2 = pl.pallas_call(
    _cls2_body,
    grid=(GRID,),
    in_specs=[_f_spec, _row_spec, _s_spec, _s_spec],
    out_specs=[_f_spec, _row_spec],
    out_shape=[
        jax.ShapeDtypeStruct((P, 1), jnp.int32),
        jax.ShapeDtypeStruct((P, C), jnp.float32),
    ],
)


def kernel(x, edge_index, target_label, W1, b1, Wa0, ba0, Wb0, bb0,
           Wa1, ba1, Wb1, bb1, Wc, bc):
    src = edge_index[0].astype(jnp.int32)
    dst = edge_index[1].astype(jnp.int32)
    bsrc, bdst, cnts = _bin_edges(src, dst)

    bsrc3 = bsrc.reshape(NW, CAP // G, G)
    bdst3 = bdst.reshape(NW, CAP // G, G)

    def agg_x(tab):
        return _agg_n(tab, bsrc3, bdst3, cnts)

    def agg_p(tab):
        return _agg_p(tab, bsrc3, bdst3, cnts)

    out = _mm_relu(agg_x(x), W1, b1.reshape(1, C))
    for (Wa, ba, Wb, bb) in ((Wa0, ba0, Wb0, bb0), (Wa1, ba1, Wb1, bb1)):
        h = _mm_relu(agg_p(out), Wa, ba.reshape(1, C))
        out = _res(out, agg_p(h), Wb, bb.reshape(1, C))
    ocls, mx, top = _cls1(agg_p(out), Wc.reshape(1, C), bc.reshape(1, 1))
    keep, opr = _cls2(ocls, out, mx, top)
    return (opr[:N], ocls[:N], target_label, keep[:N, 0].astype(bool))
